# Initial kernel scaffold; baseline (speedup 1.0000x reference)
import jax
import jax.numpy as jnp
from jax.experimental import pallas as pl


def kernel(node_feats, edge_index, graph_ids, W_lift, b_lift, Wm1, bm1, Wo1, bo1, Wm2, bm2, Wo2, bo2, Wm3, bm3, Wo3, bo3, W_read, b_read):
    raise NotImplementedError("write your pallas kernel here")


# speedup vs baseline: 3.9796x; 3.9796x over previous
import functools

import jax
import jax.numpy as jnp
from jax import lax
from jax.experimental import pallas as pl
from jax.experimental.pallas import tpu as pltpu
from jax.experimental.pallas import tpu_sc as plsc

N = 10000
E = 160000
F = 119
H = 300
HP = 304
HH = 152
V = 128
G = 10

NSUB = 16
EPT = E // NSUB
CH = 80
NCH = EPT // CH
RPT = 640
NP = NSUB * RPT

RB = 2000
NRB = N // RB


def _sc_body(m0, m1, srcr, dstr, zeros, out0, out1, src_v, dst_v, rows_v, acc, sem):
    c = lax.axis_index("c")
    s = lax.axis_index("s")
    stripe = pl.multiple_of(s * RPT, RPT)
    for zb in range(RPT // CH):
        pltpu.sync_copy(zeros, acc.at[pl.ds(stripe + zb * CH, CH)])
    pltpu.sync_copy(srcr.at[s], src_v)
    pltpu.sync_copy(dstr.at[s], dst_v)
    plsc.subcore_barrier()

    for cv, m in ((0, m0), (1, m1)):
        @pl.when(c == cv)
        def _():
            def chunk(j, carry):
                pltpu.async_copy(m.at[src_v.at[j]], rows_v, sem).wait()
                pltpu.sync_copy(rows_v, acc.at[dst_v.at[j]], add=True)
                return carry
            lax.fori_loop(0, NCH, chunk, 0)

    plsc.subcore_barrier()
    for cv, out in ((0, out0), (1, out1)):
        @pl.when(c == cv)
        def _():
            pltpu.sync_copy(acc.at[pl.ds(stripe, RPT)],
                            out.at[pl.ds(stripe, RPT)])


@functools.cache
def _sc_scatter():
    return pl.kernel(
        _sc_body,
        mesh=plsc.VectorSubcoreMesh(core_axis_name="c", subcore_axis_name="s"),
        compiler_params=pltpu.CompilerParams(use_tc_tiling_on_sc=False),
        out_type=[jax.ShapeDtypeStruct((NP, HH), jnp.float32),
                  jax.ShapeDtypeStruct((NP, HH), jnp.float32)],
        scratch_types=[
            pltpu.VMEM((NCH, CH), jnp.int32),
            pltpu.VMEM((NCH, CH), jnp.int32),
            pltpu.VMEM((CH, HH), jnp.float32),
            pltpu.VMEM_SHARED((NP, HH), jnp.float32),
            pltpu.SemaphoreType.DMA,
        ],
    )


def _dot(a, b):
    return jnp.dot(a, b, preferred_element_type=jnp.float32)


def _tc_lift_body(nf, Wl, bl, WmA, bmA, WmB, bmB, o0, o1):
    h = _dot(nf[...], Wl[...]) + bl[...]
    o0[...] = jax.nn.relu(_dot(h, WmA[...]) + bmA[...])
    o1[...] = jax.nn.relu(_dot(h, WmB[...]) + bmB[...])


def _tc_mid_body(h0, h1, WoA, WoB, bo, WmA, bmA, WmB, bmB, o0, o1):
    hh = jax.nn.relu(_dot(h0[...], WoA[...]) + _dot(h1[...], WoB[...]) + bo[...])
    o0[...] = jax.nn.relu(_dot(hh, WmA[...]) + bmA[...])
    o1[...] = jax.nn.relu(_dot(hh, WmB[...]) + bmB[...])


def _tc_read_body(h0, h1, WoA, WoB, bo, Wr, br, gid, out):
    hh = jax.nn.relu(_dot(h0[...], WoA[...]) + _dot(h1[...], WoB[...]) + bo[...])
    nl = _dot(hh, Wr[...]) + br[...]
    ids = gid[0, 0, :]
    iot = lax.broadcasted_iota(jnp.int32, (G, RB), 0)
    onehot = (ids[None, :] == iot).astype(jnp.float32)

    @pl.when(pl.program_id(0) == 0)
    def _():
        out[...] = jnp.zeros_like(out)

    out[...] += _dot(onehot, nl)


def _full(shape):
    return pl.BlockSpec(shape, lambda i: (0,) * len(shape))


def _rows(width):
    return pl.BlockSpec((RB, width), lambda i: (i, 0))


_tc_lift = pl.pallas_call(
    _tc_lift_body,
    grid=(NRB,),
    in_specs=[_rows(F), _full((F, H)), _full((1, H)),
              _full((H, HH)), _full((1, HH)), _full((H, HH)), _full((1, HH))],
    out_specs=[_rows(HH), _rows(HH)],
    out_shape=[jax.ShapeDtypeStruct((N, HH), jnp.float32)] * 2,
)

_tc_mid = pl.pallas_call(
    _tc_mid_body,
    grid=(NRB,),
    in_specs=[_rows(HH), _rows(HH),
              _full((HH, H)), _full((HH, H)), _full((1, H)),
              _full((H, HH)), _full((1, HH)), _full((H, HH)), _full((1, HH))],
    out_specs=[_rows(HH), _rows(HH)],
    out_shape=[jax.ShapeDtypeStruct((N, HH), jnp.float32)] * 2,
)

_tc_read = pl.pallas_call(
    _tc_read_body,
    grid=(NRB,),
    in_specs=[_rows(HH), _rows(HH),
              _full((HH, H)), _full((HH, H)), _full((1, H)),
              _full((H, V)), _full((1, V)),
              pl.BlockSpec((1, 1, RB), lambda i: (i, 0, 0))],
    out_specs=pl.BlockSpec((G, V), lambda i: (0, 0)),
    out_shape=jax.ShapeDtypeStruct((G, V), jnp.float32),
)


def _split_msg(Wm, bm):
    Wp = jnp.pad(Wm, ((0, 0), (0, HP - H)))
    bp = jnp.pad(bm, (0, HP - H)).reshape(1, HP)
    return Wp[:, :HH], bp[:, :HH], Wp[:, HH:], bp[:, HH:]


def _split_out(Wo):
    Wp = jnp.pad(Wo, ((0, HP - H), (0, 0)))
    return Wp[:HH, :], Wp[HH:, :]


def kernel(node_feats, edge_index, graph_ids,
           W_lift, b_lift,
           Wm1, bm1, Wo1, bo1,
           Wm2, bm2, Wo2, bo2,
           Wm3, bm3, Wo3, bo3,
           W_read, b_read):
    src = edge_index[0].reshape(NSUB, NCH, CH)
    dst = edge_index[1].reshape(NSUB, NCH, CH)
    gid = graph_ids.reshape(NRB, 1, RB)
    zeros = jnp.zeros((CH, HH), jnp.float32)

    m1A, c1A, m1B, c1B = _split_msg(Wm1, bm1)
    m2A, c2A, m2B, c2B = _split_msg(Wm2, bm2)
    m3A, c3A, m3B, c3B = _split_msg(Wm3, bm3)
    o1A, o1B = _split_out(Wo1)
    o2A, o2B = _split_out(Wo2)
    o3A, o3B = _split_out(Wo3)

    p0, p1 = _tc_lift(node_feats, W_lift, b_lift.reshape(1, H),
                      m1A, c1A, m1B, c1B)
    a0, a1 = _sc_scatter()(p0, p1, src, dst, zeros)
    p0, p1 = _tc_mid(a0, a1, o1A, o1B, bo1.reshape(1, H),
                     m2A, c2A, m2B, c2B)
    a0, a1 = _sc_scatter()(p0, p1, src, dst, zeros)
    p0, p1 = _tc_mid(a0, a1, o2A, o2B, bo2.reshape(1, H),
                     m3A, c3A, m3B, c3B)
    a0, a1 = _sc_scatter()(p0, p1, src, dst, zeros)
    logits = _tc_read(a0, a1, o3A, o3B, bo3.reshape(1, H),
                      W_read, b_read.reshape(1, V), gid)
    return logits


# speedup vs baseline: 4.8521x; 1.2192x over previous
import functools

import jax
import jax.numpy as jnp
from jax import lax
from jax.experimental import pallas as pl
from jax.experimental.pallas import tpu as pltpu
from jax.experimental.pallas import tpu_sc as plsc

N = 10000
E = 160000
F = 119
H = 300
HP = 304
HH = 152
V = 128
G = 10

NSUB = 16
EPT = E // NSUB
CH = 40
NCH = EPT // CH
NPAIR = NCH // 2
ZB = 128
RPT = 640
NP = NSUB * RPT

RB = 2000
NRB = N // RB


def _sc_body(m0, m1, srcr, dstr, zeros, out0, out1, src_v, dst_v,
             rows0, rows1, acc, sem0, sem1):
    c = lax.axis_index("c")
    s = lax.axis_index("s")
    stripe = pl.multiple_of(s * RPT, RPT)
    for zb in range(RPT // ZB):
        pltpu.sync_copy(zeros, acc.at[pl.ds(stripe + zb * ZB, ZB)])
    pltpu.sync_copy(srcr.at[s], src_v)
    pltpu.sync_copy(dstr.at[s], dst_v)
    plsc.subcore_barrier()

    for cv, m in ((0, m0), (1, m1)):
        @pl.when(c == cv)
        def _():
            pltpu.async_copy(m.at[src_v.at[0]], rows0, sem0)

            def pair(jj, carry):
                j0 = jj * 2
                pltpu.async_copy(m.at[src_v.at[j0 + 1]], rows1, sem1)
                pltpu.make_async_copy(m.at[src_v.at[j0]], rows0, sem0).wait()
                pltpu.sync_copy(rows0, acc.at[dst_v.at[j0]], add=True)

                @pl.when(jj + 1 < NPAIR)
                def _():
                    pltpu.async_copy(m.at[src_v.at[j0 + 2]], rows0, sem0)
                pltpu.make_async_copy(m.at[src_v.at[j0 + 1]], rows1, sem1).wait()
                pltpu.sync_copy(rows1, acc.at[dst_v.at[j0 + 1]], add=True)
                return carry
            lax.fori_loop(0, NPAIR, pair, 0)

    plsc.subcore_barrier()
    for cv, out in ((0, out0), (1, out1)):
        @pl.when(c == cv)
        def _():
            pltpu.sync_copy(acc.at[pl.ds(stripe, RPT)],
                            out.at[pl.ds(stripe, RPT)])


@functools.cache
def _sc_scatter():
    return pl.kernel(
        _sc_body,
        mesh=plsc.VectorSubcoreMesh(core_axis_name="c", subcore_axis_name="s"),
        compiler_params=pltpu.CompilerParams(use_tc_tiling_on_sc=False),
        out_type=[jax.ShapeDtypeStruct((NP, HH), jnp.float32),
                  jax.ShapeDtypeStruct((NP, HH), jnp.float32)],
        scratch_types=[
            pltpu.VMEM((NCH, CH), jnp.int32),
            pltpu.VMEM((NCH, CH), jnp.int32),
            pltpu.VMEM((CH, HH), jnp.float32),
            pltpu.VMEM((CH, HH), jnp.float32),
            pltpu.VMEM_SHARED((NP, HH), jnp.float32),
            pltpu.SemaphoreType.DMA,
            pltpu.SemaphoreType.DMA,
        ],
    )


def _dot(a, b):
    return jnp.dot(a, b, preferred_element_type=jnp.float32)


def _tc_lift_body(nf, Wl, bl, WmA, bmA, WmB, bmB, o0, o1):
    h = _dot(nf[...], Wl[...]) + bl[...]
    o0[...] = jax.nn.relu(_dot(h, WmA[...]) + bmA[...])
    o1[...] = jax.nn.relu(_dot(h, WmB[...]) + bmB[...])


def _tc_mid_body(h0, h1, WoA, WoB, bo, WmA, bmA, WmB, bmB, o0, o1):
    hh = jax.nn.relu(_dot(h0[...], WoA[...]) + _dot(h1[...], WoB[...]) + bo[...])
    o0[...] = jax.nn.relu(_dot(hh, WmA[...]) + bmA[...])
    o1[...] = jax.nn.relu(_dot(hh, WmB[...]) + bmB[...])


def _tc_read_body(h0, h1, WoA, WoB, bo, Wr, br, gid, out):
    hh = jax.nn.relu(_dot(h0[...], WoA[...]) + _dot(h1[...], WoB[...]) + bo[...])
    nl = _dot(hh, Wr[...]) + br[...]
    ids = gid[0, 0, :]
    iot = lax.broadcasted_iota(jnp.int32, (G, RB), 0)
    onehot = (ids[None, :] == iot).astype(jnp.float32)

    @pl.when(pl.program_id(0) == 0)
    def _():
        out[...] = jnp.zeros_like(out)

    out[...] += _dot(onehot, nl)


def _full(shape):
    return pl.BlockSpec(shape, lambda i: (0,) * len(shape))


def _rows(width):
    return pl.BlockSpec((RB, width), lambda i: (i, 0))


_tc_lift = pl.pallas_call(
    _tc_lift_body,
    grid=(NRB,),
    in_specs=[_rows(F), _full((F, H)), _full((1, H)),
              _full((H, HH)), _full((1, HH)), _full((H, HH)), _full((1, HH))],
    out_specs=[_rows(HH), _rows(HH)],
    out_shape=[jax.ShapeDtypeStruct((N, HH), jnp.float32)] * 2,
)

_tc_mid = pl.pallas_call(
    _tc_mid_body,
    grid=(NRB,),
    in_specs=[_rows(HH), _rows(HH),
              _full((HH, H)), _full((HH, H)), _full((1, H)),
              _full((H, HH)), _full((1, HH)), _full((H, HH)), _full((1, HH))],
    out_specs=[_rows(HH), _rows(HH)],
    out_shape=[jax.ShapeDtypeStruct((N, HH), jnp.float32)] * 2,
)

_tc_read = pl.pallas_call(
    _tc_read_body,
    grid=(NRB,),
    in_specs=[_rows(HH), _rows(HH),
              _full((HH, H)), _full((HH, H)), _full((1, H)),
              _full((H, V)), _full((1, V)),
              pl.BlockSpec((1, 1, RB), lambda i: (i, 0, 0))],
    out_specs=pl.BlockSpec((G, V), lambda i: (0, 0)),
    out_shape=jax.ShapeDtypeStruct((G, V), jnp.float32),
)


def _split_msg(Wm, bm):
    Wp = jnp.pad(Wm, ((0, 0), (0, HP - H)))
    bp = jnp.pad(bm, (0, HP - H)).reshape(1, HP)
    return Wp[:, :HH], bp[:, :HH], Wp[:, HH:], bp[:, HH:]


def _split_out(Wo):
    Wp = jnp.pad(Wo, ((0, HP - H), (0, 0)))
    return Wp[:HH, :], Wp[HH:, :]


def kernel(node_feats, edge_index, graph_ids,
           W_lift, b_lift,
           Wm1, bm1, Wo1, bo1,
           Wm2, bm2, Wo2, bo2,
           Wm3, bm3, Wo3, bo3,
           W_read, b_read):
    src = edge_index[0].reshape(NSUB, NCH, CH)
    dst = edge_index[1].reshape(NSUB, NCH, CH)
    gid = graph_ids.reshape(NRB, 1, RB)
    zeros = jnp.zeros((ZB, HH), jnp.float32)

    m1A, c1A, m1B, c1B = _split_msg(Wm1, bm1)
    m2A, c2A, m2B, c2B = _split_msg(Wm2, bm2)
    m3A, c3A, m3B, c3B = _split_msg(Wm3, bm3)
    o1A, o1B = _split_out(Wo1)
    o2A, o2B = _split_out(Wo2)
    o3A, o3B = _split_out(Wo3)

    p0, p1 = _tc_lift(node_feats, W_lift, b_lift.reshape(1, H),
                      m1A, c1A, m1B, c1B)
    a0, a1 = _sc_scatter()(p0, p1, src, dst, zeros)
    p0, p1 = _tc_mid(a0, a1, o1A, o1B, bo1.reshape(1, H),
                     m2A, c2A, m2B, c2B)
    a0, a1 = _sc_scatter()(p0, p1, src, dst, zeros)
    p0, p1 = _tc_mid(a0, a1, o2A, o2B, bo2.reshape(1, H),
                     m3A, c3A, m3B, c3B)
    a0, a1 = _sc_scatter()(p0, p1, src, dst, zeros)
    logits = _tc_read(a0, a1, o3A, o3B, bo3.reshape(1, H),
                      W_read, b_read.reshape(1, V), gid)
    return logits
